# baseline (speedup 1.0000x reference)
import functools

import jax
import jax.numpy as jnp
from jax import lax
from jax.experimental import pallas as pl
from jax.experimental.pallas import tpu as pltpu
from jax.experimental.pallas import tpu_sc as plsc

N = 10000
E = 320000
D = 128
ALPHA = 0.5

NC = 2
NS = 16
CHUNK = 128
NCHUNK = E // CHUNK
K = 8
NCHUNK_PAD = 2504
NBLK = NCHUNK_PAD // K
ROWS_PER_TILE = 640
ROW_STRIDE = 624
CNT_N = 10240
CNT_PER_TILE = CNT_N // NS


def _sc_agg_body(with_counts, *refs):
    if with_counts:
        (h_hbm, eidx_hbm, zrows_hbm, zcnt_hbm, out_hbm, cnt_hbm,
         ibuf, rows0, rows1, acc_sh,
         sem_g0, sem_g1, sem_s0, sem_s1, sem_i,
         ones_v, cnt_sh, sem_c) = refs
    else:
        (h_hbm, eidx_hbm, zrows_hbm, out_hbm,
         ibuf, rows0, rows1, acc_sh,
         sem_g0, sem_g1, sem_s0, sem_s1, sem_i) = refs
    c = lax.axis_index("c")
    s = lax.axis_index("s")
    pltpu.sync_copy(zrows_hbm, acc_sh.at[pl.ds(s * ROW_STRIDE, ROWS_PER_TILE)])
    if with_counts:
        pltpu.sync_copy(zcnt_hbm, cnt_sh.at[pl.ds(s * CNT_PER_TILE, CNT_PER_TILE)])
        for k in range(CHUNK // 16):
            ones_v[pl.ds(k * 16, 16)] = jnp.ones((16,), jnp.float32)
    plsc.subcore_barrier()
    lo_b = (s * NBLK) // NS
    hi_b = ((s + 1) * NBLK) // NS
    rows = (rows0, rows1)
    sem_g = (sem_g0, sem_g1)
    sem_s = (sem_s0, sem_s1)

    def _gather(bp, k, p):
        return pltpu.make_async_copy(h_hbm.at[ibuf.at[bp, c, k]], rows[p], sem_g[p])

    def _scatter(bp, k, p):
        return pltpu.make_async_copy(rows[p], acc_sh.at[ibuf.at[bp, 1 - c, k]],
                                     sem_s[p])

    def _cscatter(bp, k):
        return pltpu.make_async_copy(ones_v, cnt_sh.at[ibuf.at[bp, 1 - c, k]],
                                     sem_c)

    pltpu.sync_copy(eidx_hbm.at[:, pl.ds(lo_b * K, K), :], ibuf.at[0])
    _gather(0, 0, 0).start()

    def blk(jb, carry):
        bp = (jb - lo_b) % 2
        base = jb * K
        for k in range(K):
            p = k % 2
            if k == 0:
                @pl.when(jb > lo_b)
                def _():
                    _scatter(1 - bp, K - 1, 1).wait()
                    if with_counts:
                        for kk in range(K):
                            _cscatter(1 - bp, kk).wait()

                @pl.when(jb + 1 < hi_b)
                def _():
                    pltpu.async_copy(eidx_hbm.at[:, pl.ds((jb + 1) * K, K), :],
                                     ibuf.at[1 - bp], sem_i)
            else:
                @pl.when(base + k - 1 < NCHUNK)
                def _():
                    _scatter(bp, k - 1, 1 - p).wait()
            if k + 1 < K:
                @pl.when(base + k + 1 < NCHUNK)
                def _():
                    _gather(bp, k + 1, 1 - p).start()
            else:
                @pl.when(jb + 1 < hi_b)
                def _():
                    pltpu.make_async_copy(
                        eidx_hbm.at[:, pl.ds((jb + 1) * K, K), :],
                        ibuf.at[1 - bp], sem_i).wait()
                    _gather(1 - bp, 0, 1 - p).start()

            @pl.when(base + k < NCHUNK)
            def _():
                _gather(bp, k, p).wait()
                _scatter(bp, k, p).start(add=True)
                if with_counts:
                    _cscatter(bp, k).start(add=True)
        return carry

    lax.fori_loop(lo_b, hi_b, blk, 0)
    @pl.when(hi_b * K <= NCHUNK)
    def _():
        _scatter((hi_b - 1 - lo_b) % 2, K - 1, 1).wait()

    if with_counts:
        last_bp = (hi_b - 1 - lo_b) % 2
        for kk in range(K):
            @pl.when((hi_b - 1) * K + kk < NCHUNK)
            def _():
                _cscatter(last_bp, kk).wait()

    plsc.subcore_barrier()
    pltpu.sync_copy(
        acc_sh.at[pl.ds(s * ROW_STRIDE, ROWS_PER_TILE)],
        out_hbm.at[c, pl.ds(s * ROW_STRIDE, ROWS_PER_TILE)],
    )
    if with_counts:
        pltpu.sync_copy(
            cnt_sh.at[pl.ds(s * CNT_PER_TILE, CNT_PER_TILE)],
            cnt_hbm.at[c, pl.ds(s * CNT_PER_TILE, CNT_PER_TILE)],
        )


_AGG_SCRATCH = [
    pltpu.VMEM((2, NC, K, CHUNK), jnp.int32),
    pltpu.VMEM((CHUNK, D), jnp.float32),
    pltpu.VMEM((CHUNK, D), jnp.float32),
    pltpu.VMEM_SHARED((N, D), jnp.float32),
    pltpu.SemaphoreType.DMA,
    pltpu.SemaphoreType.DMA,
    pltpu.SemaphoreType.DMA,
    pltpu.SemaphoreType.DMA,
    pltpu.SemaphoreType.DMA,
]


@jax.jit
def _sc_agg(h, eidx3, zrows):
    mesh = plsc.VectorSubcoreMesh(core_axis_name="c", subcore_axis_name="s")
    return pl.kernel(
        functools.partial(_sc_agg_body, False),
        out_type=jax.ShapeDtypeStruct((NC, N, D), jnp.float32),
        mesh=mesh,
        scratch_types=list(_AGG_SCRATCH),
    )(h, eidx3, zrows)


@jax.jit
def _sc_agg_cnt(h, eidx3, zrows, zcnt):
    mesh = plsc.VectorSubcoreMesh(core_axis_name="c", subcore_axis_name="s")
    return pl.kernel(
        functools.partial(_sc_agg_body, True),
        out_type=(jax.ShapeDtypeStruct((NC, N, D), jnp.float32),
                  jax.ShapeDtypeStruct((NC, CNT_N), jnp.float32)),
        mesh=mesh,
        scratch_types=list(_AGG_SCRATCH) + [
            pltpu.VMEM((CHUNK,), jnp.float32),
            pltpu.VMEM_SHARED((CNT_N,), jnp.float32),
            pltpu.SemaphoreType.DMA,
        ],
    )(h, eidx3, zrows, zcnt)


BN = 1000


_DN_T = (((1,), (1,)), ((), ()))


def _tc_layer_body(l, has_m, final, *refs):
    if final:
        (h_ref, a0_ref, a1_ref, cd_ref, cs_ref, m_ref,
         ws_ref, bs_ref, w1_ref, b1_ref, w2_ref, b2_ref,
         wo_ref, bo_ref, out_ref) = refs
    elif has_m:
        (h_ref, a0_ref, a1_ref, cd_ref, cs_ref, m_ref,
         ws_ref, bs_ref, w1_ref, b1_ref, w2_ref, b2_ref,
         hout_ref, mout_ref) = refs
    else:
        (h_ref, a0_ref, a1_ref, cd_ref, cs_ref,
         ws_ref, bs_ref, w1_ref, b1_ref, w2_ref, b2_ref,
         hout_ref) = refs
    inv_d = 1.0 / jnp.maximum(cd_ref[...], 1.0)
    inv_s = 1.0 / jnp.maximum(cs_ref[...], 1.0)
    y = lax.dot_general(h_ref[...], ws_ref[0], _DN_T,
                        preferred_element_type=jnp.float32)
    y += bs_ref[l]
    y += (1.0 - ALPHA) * (
        lax.dot_general(a0_ref[0] * inv_d, w1_ref[0], _DN_T,
                        preferred_element_type=jnp.float32) + b1_ref[l])
    y += ALPHA * (
        lax.dot_general(a1_ref[0] * inv_s, w2_ref[0], _DN_T,
                        preferred_element_type=jnp.float32) + b2_ref[l])
    h_new = jnp.maximum(y, 0.0)
    if final:
        m_new = jnp.maximum(m_ref[...], h_new)
        out_ref[...] = (
            lax.dot_general(m_new, wo_ref[...], _DN_T,
                            preferred_element_type=jnp.float32) + bo_ref[...])
    elif has_m:
        hout_ref[...] = h_new
        mout_ref[...] = jnp.maximum(m_ref[...], h_new)
    else:
        hout_ref[...] = h_new


_F_SPEC = pl.BlockSpec((BN, D), lambda i: (i, 0))
_A0_SPEC = pl.BlockSpec((1, BN, D), lambda i: (0, i, 0))
_A1_SPEC = pl.BlockSpec((1, BN, D), lambda i: (1, i, 0))
_W2_SPEC = pl.BlockSpec((D, D), lambda i: (0, 0))
_B1_SPEC = pl.BlockSpec((1, D), lambda i: (0, 0))
_C_SPEC = pl.BlockSpec((BN, 1), lambda i: (i, 0))
_FOUT = jax.ShapeDtypeStruct((N, D), jnp.float32)


def _lw_specs(l):
    w = pl.BlockSpec((1, D, D), lambda i, l=l: (l, 0, 0))
    b = pl.BlockSpec((3, D), lambda i: (0, 0))
    return [w, b, w, b, w, b]


@functools.partial(jax.jit, static_argnums=(0,))
def _tc_layer_first(l, h, agg, cd, cs, Ws, bs, W1, b1, W2, b2):
    return pl.pallas_call(
        functools.partial(_tc_layer_body, l, False, False),
        grid=(N // BN,),
        in_specs=[_F_SPEC, _A0_SPEC, _A1_SPEC, _C_SPEC, _C_SPEC] + _lw_specs(l),
        out_specs=_F_SPEC,
        out_shape=_FOUT,
    )(h, agg, agg, cd, cs, Ws, bs, W1, b1, W2, b2)


@functools.partial(jax.jit, static_argnums=(0,))
def _tc_layer_mid(l, h, agg, cd, cs, m, Ws, bs, W1, b1, W2, b2):
    return pl.pallas_call(
        functools.partial(_tc_layer_body, l, True, False),
        grid=(N // BN,),
        in_specs=[_F_SPEC, _A0_SPEC, _A1_SPEC, _C_SPEC, _C_SPEC, _F_SPEC]
        + _lw_specs(l),
        out_specs=(_F_SPEC, _F_SPEC),
        out_shape=(_FOUT, _FOUT),
    )(h, agg, agg, cd, cs, m, Ws, bs, W1, b1, W2, b2)


@functools.partial(jax.jit, static_argnums=(0,))
def _tc_layer_last(l, h, agg, cd, cs, m, Ws, bs, W1, b1, W2, b2, Wo, bo):
    return pl.pallas_call(
        functools.partial(_tc_layer_body, l, True, True),
        grid=(N // BN,),
        in_specs=[_F_SPEC, _A0_SPEC, _A1_SPEC, _C_SPEC, _C_SPEC, _F_SPEC]
        + _lw_specs(l) + [_W2_SPEC, _B1_SPEC],
        out_specs=_F_SPEC,
        out_shape=_FOUT,
    )(h, agg, agg, cd, cs, m, Ws, bs, W1, b1, W2, b2, Wo, bo)


def kernel(x, edge_index, W_self, b_self, W_s2d, b_s2d, W_d2s, b_d2s, W_out, b_out):
    eidx3 = jnp.pad(edge_index.reshape(2, NCHUNK, CHUNK),
                    ((0, 0), (0, NCHUNK_PAD - NCHUNK), (0, 0)))
    zrows = jnp.zeros((ROWS_PER_TILE, D), jnp.float32)
    zcnt = jnp.zeros((CNT_PER_TILE,), jnp.float32)

    agg, cnts = _sc_agg_cnt(x, eidx3, zrows, zcnt)
    cd = cnts[0, :N].reshape(N, 1)
    cs = cnts[1, :N].reshape(N, 1)

    lw = (W_self, b_self, W_s2d, b_s2d, W_d2s, b_d2s)
    h1 = _tc_layer_first(0, x, agg, cd, cs, *lw)
    agg = _sc_agg(h1, eidx3, zrows)
    h2, m2 = _tc_layer_mid(1, h1, agg, cd, cs, h1, *lw)
    agg = _sc_agg(h2, eidx3, zrows)
    return _tc_layer_last(2, h2, agg, cd, cs, m2, *lw,
                          W_out, b_out.reshape(1, D))


# speedup vs baseline: 1.0067x; 1.0067x over previous
import functools

import jax
import jax.numpy as jnp
from jax import lax
from jax.experimental import pallas as pl
from jax.experimental.pallas import tpu as pltpu
from jax.experimental.pallas import tpu_sc as plsc

N = 10000
E = 320000
D = 128
ALPHA = 0.5

NC = 2
NS = 16
CHUNK = 128
NCHUNK = E // CHUNK
K = 8
NCHUNK_PAD = 2504
NBLK = NCHUNK_PAD // K
ROWS_PER_TILE = 640
ROW_STRIDE = 624
CNT_N = 10240
CNT_PER_TILE = CNT_N // NS


def _sc_agg_body(with_counts, *refs):
    if with_counts:
        (h_hbm, eidx_hbm, zrows_hbm, zcnt_hbm, out_hbm, cnt_hbm,
         ibuf, rows0, rows1, acc_sh,
         sem_g0, sem_g1, sem_s0, sem_s1, sem_i,
         ones_v, cnt_sh, sem_c) = refs
    else:
        (h_hbm, eidx_hbm, zrows_hbm, out_hbm,
         ibuf, rows0, rows1, acc_sh,
         sem_g0, sem_g1, sem_s0, sem_s1, sem_i) = refs
    c = lax.axis_index("c")
    s = lax.axis_index("s")
    lo_b = (s * NBLK) // NS
    hi_b = ((s + 1) * NBLK) // NS
    rows = (rows0, rows1)
    sem_g = (sem_g0, sem_g1)
    sem_s = (sem_s0, sem_s1)

    def _gather(bp, k, p):
        return pltpu.make_async_copy(h_hbm.at[ibuf.at[bp, c, k]], rows[p], sem_g[p])

    def _scatter(bp, k, p):
        return pltpu.make_async_copy(rows[p], acc_sh.at[ibuf.at[bp, 1 - c, k]],
                                     sem_s[p])

    def _cscatter(bp, k):
        return pltpu.make_async_copy(ones_v, cnt_sh.at[ibuf.at[bp, 1 - c, k]],
                                     sem_c)

    pltpu.sync_copy(eidx_hbm.at[:, pl.ds(lo_b * K, K), :], ibuf.at[0])
    _gather(0, 0, 0).start()
    pltpu.sync_copy(zrows_hbm, acc_sh.at[pl.ds(s * ROW_STRIDE, ROWS_PER_TILE)])
    if with_counts:
        pltpu.sync_copy(zcnt_hbm, cnt_sh.at[pl.ds(s * CNT_PER_TILE, CNT_PER_TILE)])
        for k in range(CHUNK // 16):
            ones_v[pl.ds(k * 16, 16)] = jnp.ones((16,), jnp.float32)
    plsc.subcore_barrier()

    def blk(jb, carry):
        bp = (jb - lo_b) % 2
        base = jb * K
        for k in range(K):
            p = k % 2
            if k == 0:
                @pl.when(jb > lo_b)
                def _():
                    _scatter(1 - bp, K - 1, 1).wait()
                    if with_counts:
                        for kk in range(K):
                            _cscatter(1 - bp, kk).wait()

                @pl.when(jb + 1 < hi_b)
                def _():
                    pltpu.async_copy(eidx_hbm.at[:, pl.ds((jb + 1) * K, K), :],
                                     ibuf.at[1 - bp], sem_i)
            else:
                @pl.when(base + k - 1 < NCHUNK)
                def _():
                    _scatter(bp, k - 1, 1 - p).wait()
            if k + 1 < K:
                @pl.when(base + k + 1 < NCHUNK)
                def _():
                    _gather(bp, k + 1, 1 - p).start()
            else:
                @pl.when(jb + 1 < hi_b)
                def _():
                    pltpu.make_async_copy(
                        eidx_hbm.at[:, pl.ds((jb + 1) * K, K), :],
                        ibuf.at[1 - bp], sem_i).wait()
                    _gather(1 - bp, 0, 1 - p).start()

            @pl.when(base + k < NCHUNK)
            def _():
                _gather(bp, k, p).wait()
                _scatter(bp, k, p).start(add=True)
                if with_counts:
                    _cscatter(bp, k).start(add=True)
        return carry

    lax.fori_loop(lo_b, hi_b, blk, 0)
    @pl.when(hi_b * K <= NCHUNK)
    def _():
        _scatter((hi_b - 1 - lo_b) % 2, K - 1, 1).wait()

    if with_counts:
        last_bp = (hi_b - 1 - lo_b) % 2
        for kk in range(K):
            @pl.when((hi_b - 1) * K + kk < NCHUNK)
            def _():
                _cscatter(last_bp, kk).wait()

    plsc.subcore_barrier()
    pltpu.sync_copy(
        acc_sh.at[pl.ds(s * ROW_STRIDE, ROWS_PER_TILE)],
        out_hbm.at[c, pl.ds(s * ROW_STRIDE, ROWS_PER_TILE)],
    )
    if with_counts:
        pltpu.sync_copy(
            cnt_sh.at[pl.ds(s * CNT_PER_TILE, CNT_PER_TILE)],
            cnt_hbm.at[c, pl.ds(s * CNT_PER_TILE, CNT_PER_TILE)],
        )


_AGG_SCRATCH = [
    pltpu.VMEM((2, NC, K, CHUNK), jnp.int32),
    pltpu.VMEM((CHUNK, D), jnp.float32),
    pltpu.VMEM((CHUNK, D), jnp.float32),
    pltpu.VMEM_SHARED((N, D), jnp.float32),
    pltpu.SemaphoreType.DMA,
    pltpu.SemaphoreType.DMA,
    pltpu.SemaphoreType.DMA,
    pltpu.SemaphoreType.DMA,
    pltpu.SemaphoreType.DMA,
]


@jax.jit
def _sc_agg(h, eidx3, zrows):
    mesh = plsc.VectorSubcoreMesh(core_axis_name="c", subcore_axis_name="s")
    return pl.kernel(
        functools.partial(_sc_agg_body, False),
        out_type=jax.ShapeDtypeStruct((NC, N, D), jnp.float32),
        mesh=mesh,
        scratch_types=list(_AGG_SCRATCH),
    )(h, eidx3, zrows)


@jax.jit
def _sc_agg_cnt(h, eidx3, zrows, zcnt):
    mesh = plsc.VectorSubcoreMesh(core_axis_name="c", subcore_axis_name="s")
    return pl.kernel(
        functools.partial(_sc_agg_body, True),
        out_type=(jax.ShapeDtypeStruct((NC, N, D), jnp.float32),
                  jax.ShapeDtypeStruct((NC, CNT_N), jnp.float32)),
        mesh=mesh,
        scratch_types=list(_AGG_SCRATCH) + [
            pltpu.VMEM((CHUNK,), jnp.float32),
            pltpu.VMEM_SHARED((CNT_N,), jnp.float32),
            pltpu.SemaphoreType.DMA,
        ],
    )(h, eidx3, zrows, zcnt)


BN = 1000


_DN_T = (((1,), (1,)), ((), ()))


def _tc_layer_body(l, has_m, final, *refs):
    if final:
        (h_ref, a0_ref, a1_ref, cd_ref, cs_ref, m_ref,
         ws_ref, bs_ref, w1_ref, b1_ref, w2_ref, b2_ref,
         wo_ref, bo_ref, out_ref) = refs
    elif has_m:
        (h_ref, a0_ref, a1_ref, cd_ref, cs_ref, m_ref,
         ws_ref, bs_ref, w1_ref, b1_ref, w2_ref, b2_ref,
         hout_ref, mout_ref) = refs
    else:
        (h_ref, a0_ref, a1_ref, cd_ref, cs_ref,
         ws_ref, bs_ref, w1_ref, b1_ref, w2_ref, b2_ref,
         hout_ref) = refs
    inv_d = 1.0 / jnp.maximum(cd_ref[...], 1.0)
    inv_s = 1.0 / jnp.maximum(cs_ref[...], 1.0)
    y = lax.dot_general(h_ref[...], ws_ref[0], _DN_T,
                        preferred_element_type=jnp.float32)
    y += bs_ref[l]
    y += (1.0 - ALPHA) * (
        lax.dot_general(a0_ref[0] * inv_d, w1_ref[0], _DN_T,
                        preferred_element_type=jnp.float32) + b1_ref[l])
    y += ALPHA * (
        lax.dot_general(a1_ref[0] * inv_s, w2_ref[0], _DN_T,
                        preferred_element_type=jnp.float32) + b2_ref[l])
    h_new = jnp.maximum(y, 0.0)
    if final:
        m_new = jnp.maximum(m_ref[...], h_new)
        out_ref[...] = (
            lax.dot_general(m_new, wo_ref[...], _DN_T,
                            preferred_element_type=jnp.float32) + bo_ref[...])
    elif has_m:
        hout_ref[...] = h_new
        mout_ref[...] = jnp.maximum(m_ref[...], h_new)
    else:
        hout_ref[...] = h_new


_F_SPEC = pl.BlockSpec((BN, D), lambda i: (i, 0))
_A0_SPEC = pl.BlockSpec((1, BN, D), lambda i: (0, i, 0))
_A1_SPEC = pl.BlockSpec((1, BN, D), lambda i: (1, i, 0))
_W2_SPEC = pl.BlockSpec((D, D), lambda i: (0, 0))
_B1_SPEC = pl.BlockSpec((1, D), lambda i: (0, 0))
_C_SPEC = pl.BlockSpec((BN, 1), lambda i: (i, 0))
_FOUT = jax.ShapeDtypeStruct((N, D), jnp.float32)


def _lw_specs(l):
    w = pl.BlockSpec((1, D, D), lambda i, l=l: (l, 0, 0))
    b = pl.BlockSpec((3, D), lambda i: (0, 0))
    return [w, b, w, b, w, b]


@functools.partial(jax.jit, static_argnums=(0,))
def _tc_layer_first(l, h, agg, cd, cs, Ws, bs, W1, b1, W2, b2):
    return pl.pallas_call(
        functools.partial(_tc_layer_body, l, False, False),
        grid=(N // BN,),
        in_specs=[_F_SPEC, _A0_SPEC, _A1_SPEC, _C_SPEC, _C_SPEC] + _lw_specs(l),
        out_specs=_F_SPEC,
        out_shape=_FOUT,
    )(h, agg, agg, cd, cs, Ws, bs, W1, b1, W2, b2)


@functools.partial(jax.jit, static_argnums=(0,))
def _tc_layer_mid(l, h, agg, cd, cs, m, Ws, bs, W1, b1, W2, b2):
    return pl.pallas_call(
        functools.partial(_tc_layer_body, l, True, False),
        grid=(N // BN,),
        in_specs=[_F_SPEC, _A0_SPEC, _A1_SPEC, _C_SPEC, _C_SPEC, _F_SPEC]
        + _lw_specs(l),
        out_specs=(_F_SPEC, _F_SPEC),
        out_shape=(_FOUT, _FOUT),
    )(h, agg, agg, cd, cs, m, Ws, bs, W1, b1, W2, b2)


@functools.partial(jax.jit, static_argnums=(0,))
def _tc_layer_last(l, h, agg, cd, cs, m, Ws, bs, W1, b1, W2, b2, Wo, bo):
    return pl.pallas_call(
        functools.partial(_tc_layer_body, l, True, True),
        grid=(N // BN,),
        in_specs=[_F_SPEC, _A0_SPEC, _A1_SPEC, _C_SPEC, _C_SPEC, _F_SPEC]
        + _lw_specs(l) + [_W2_SPEC, _B1_SPEC],
        out_specs=_F_SPEC,
        out_shape=_FOUT,
    )(h, agg, agg, cd, cs, m, Ws, bs, W1, b1, W2, b2, Wo, bo)


def kernel(x, edge_index, W_self, b_self, W_s2d, b_s2d, W_d2s, b_d2s, W_out, b_out):
    eidx3 = jnp.pad(edge_index.reshape(2, NCHUNK, CHUNK),
                    ((0, 0), (0, NCHUNK_PAD - NCHUNK), (0, 0)))
    zrows = jnp.zeros((ROWS_PER_TILE, D), jnp.float32)
    zcnt = jnp.zeros((CNT_PER_TILE,), jnp.float32)

    agg, cnts = _sc_agg_cnt(x, eidx3, zrows, zcnt)
    cd = cnts[0, :N].reshape(N, 1)
    cs = cnts[1, :N].reshape(N, 1)

    lw = (W_self, b_self, W_s2d, b_s2d, W_d2s, b_d2s)
    h1 = _tc_layer_first(0, x, agg, cd, cs, *lw)
    agg = _sc_agg(h1, eidx3, zrows)
    h2, m2 = _tc_layer_mid(1, h1, agg, cd, cs, h1, *lw)
    agg = _sc_agg(h2, eidx3, zrows)
    return _tc_layer_last(2, h2, agg, cd, cs, m2, *lw,
                          W_out, b_out.reshape(1, D))
